# baseline (speedup 1.0000x reference)
import functools

import jax
import jax.numpy as jnp
from jax import lax
from jax.experimental import pallas as pl
from jax.experimental.pallas import tpu as pltpu
from jax.experimental.pallas import tpu_sc as plsc

B, U, NUM_UTYPE, NUM_UFEAT, EMB_DIM = 4096, 50, 1000000, 26, 32
BU = B * U

NC, NS = 2, 16
NW = NC * NS
LANES = 128
PACK = LANES // EMB_DIM


TBLK = 16384
QBLK = TBLK // PACK
NTBLK = pl.cdiv(NUM_UTYPE, TBLK)
RM_PAD_ROWS = NTBLK * QBLK


def _eye(n):
    a = lax.broadcasted_iota(jnp.int32, (n, n), 0)
    b = lax.broadcasted_iota(jnp.int32, (n, n), 1)
    return (a == b).astype(jnp.float32)


def _tr_body(in_ref, out_ref):
    x = in_ref[...]
    xs = jnp.concatenate(
        [x[:, q * QBLK:(q + 1) * QBLK] for q in range(PACK)], axis=0)
    out_ref[...] = jax.lax.dot_general(
        xs, _eye(LANES), dimension_numbers=(((0,), (0,)), ((), ())),
        preferred_element_type=jnp.float32,
    )


@jax.jit
def _tc_transpose(tableT):
    return pl.pallas_call(
        _tr_body,
        grid=(NTBLK,),
        in_specs=[pl.BlockSpec((EMB_DIM, TBLK), lambda k: (0, k))],
        out_specs=pl.BlockSpec((QBLK, LANES), lambda k: (k, 0)),
        out_shape=jax.ShapeDtypeStruct((RM_PAD_ROWS, LANES), jnp.float32),
    )(tableT)


ROWS_PER_W = BU // NW
CHUNK = 128
NCHUNK = ROWS_PER_W // CHUNK


def _sc_gather_body(table_hbm, idx_hbm, out_hbm, idx_v,
                    ridx0, ridx1, g0, g1, sem0, sem1):
    wid = lax.axis_index("s") * NC + lax.axis_index("c")
    base = wid * ROWS_PER_W
    pltpu.sync_copy(idx_hbm.at[pl.ds(wid * NCHUNK, NCHUNK)], idx_v)

    def start(j, ridx, buf, sem):
        for m in range(CHUNK // 16):
            iv = idx_v[j, pl.ds(m * 16, 16)]
            r32 = (lax.shift_left(lax.shift_right_logical(iv, 14), 14)
                   | lax.shift_left(iv & 4095, 2)
                   | (lax.shift_right_logical(iv, 12) & 3))
            ridx[pl.ds(m * 16, 16)] = r32
        pltpu.make_async_copy(table_hbm.at[ridx], buf, sem).start()

    def drain(j, ridx, buf, sem):
        pltpu.make_async_copy(table_hbm.at[ridx], buf, sem).wait()
        pltpu.sync_copy(buf, out_hbm.at[pl.ds(base + j * CHUNK, CHUNK)])

    start(0, ridx0, g0, sem0)
    start(1, ridx1, g1, sem1)

    def body(h, carry):
        j0 = 2 * h
        drain(j0, ridx0, g0, sem0)

        @pl.when(j0 + 2 < NCHUNK)
        def _():
            start(j0 + 2, ridx0, g0, sem0)

        drain(j0 + 1, ridx1, g1, sem1)

        @pl.when(j0 + 3 < NCHUNK)
        def _():
            start(j0 + 3, ridx1, g1, sem1)

        return carry

    lax.fori_loop(0, NCHUNK // 2, body, 0)


@jax.jit
def _sc_gather(tableRM, idx2):
    mesh = plsc.VectorSubcoreMesh(core_axis_name="c", subcore_axis_name="s")
    fn = pl.kernel(
        _sc_gather_body,
        out_type=jax.ShapeDtypeStruct((BU, EMB_DIM), jnp.float32),
        mesh=mesh,
        scratch_types=[
            pltpu.VMEM((NCHUNK, CHUNK), jnp.int32),
            pltpu.VMEM((CHUNK,), jnp.int32),
            pltpu.VMEM((CHUNK,), jnp.int32),
            pltpu.VMEM((CHUNK, EMB_DIM), jnp.float32),
            pltpu.VMEM((CHUNK, EMB_DIM), jnp.float32),
            pltpu.SemaphoreType.DMA,
            pltpu.SemaphoreType.DMA,
        ],
        compiler_params=pltpu.CompilerParams(use_tc_tiling_on_sc=False),
    )
    return fn(tableRM.reshape(RM_PAD_ROWS * PACK, EMB_DIM), idx2)


CB = 256
GR = CB // PACK
NG = B // CB


def _cmb_body(g_ref, u_ref, v_ref, gn_ref, b_ref, out_ref):
    v = v_ref[...]
    scale = gn_ref[0, 0] * lax.rsqrt(jnp.sum(v * v))
    w = v * scale
    bias = b_ref[...]
    eye = _eye(GR)
    ai = lax.broadcasted_iota(jnp.int32, (CB, CB), 0)
    bi = lax.broadcasted_iota(jnp.int32, (CB, CB), 1)
    perm = (bi == ((ai & (GR - 1)) * PACK
                   + lax.shift_right_logical(ai, 6))).astype(jnp.float32)
    for u in range(U):
        x = u_ref[:, u, :]
        mm = jax.lax.dot_general(
            w, x, dimension_numbers=(((1,), (0,)), ((), ())),
            preferred_element_type=jnp.float32,
        ) + bias
        xg = g_ref[u, 0]
        xgT = jax.lax.dot_general(
            xg, eye, dimension_numbers=(((0,), (0,)), ((), ())),
            preferred_element_type=jnp.float32,
        )
        gtp = jnp.concatenate(
            [xgT[q * EMB_DIM:(q + 1) * EMB_DIM, :] for q in range(PACK)],
            axis=1)
        gt = jax.lax.dot_general(
            gtp, perm, dimension_numbers=(((1,), (0,)), ((), ())),
            preferred_element_type=jnp.float32,
        )
        out_ref[u] = jnp.concatenate([gt, mm], axis=0)


@jax.jit
def _tc_combine(gathered4, ufeatT, V, g2, bcol):
    return pl.pallas_call(
        _cmb_body,
        grid=(B // CB,),
        in_specs=[
            pl.BlockSpec((U, 1, GR, LANES), lambda c: (0, c, 0, 0)),
            pl.BlockSpec((NUM_UFEAT, U, CB), lambda c: (0, 0, c)),
            pl.BlockSpec((EMB_DIM, NUM_UFEAT), lambda c: (0, 0)),
            pl.BlockSpec((1, 1), lambda c: (0, 0), memory_space=pltpu.SMEM),
            pl.BlockSpec((EMB_DIM, 1), lambda c: (0, 0)),
        ],
        out_specs=pl.BlockSpec((U, 2 * EMB_DIM, CB), lambda c: (0, 0, c)),
        out_shape=jax.ShapeDtypeStruct((U, 2 * EMB_DIM, B), jnp.float32),
    )(gathered4, ufeatT, V, g2, bcol)


def kernel(utype, ufeat, table, V, g, b):
    tableT = table.T
    ufeatT = jnp.transpose(ufeat, (2, 1, 0))
    idxT = utype.T.astype(jnp.int32)
    idxf = lax.bitcast_convert_type(idxT, jnp.float32)
    idx_re = lax.bitcast_convert_type(
        idxf.reshape(BU // CHUNK, CHUNK), jnp.int32)

    tableRM = _tc_transpose(tableT)
    gathered = _sc_gather(tableRM, idx_re)
    g2 = jnp.reshape(g, (1, 1)).astype(jnp.float32)
    bcol = jnp.reshape(b, (EMB_DIM, 1))
    outT = _tc_combine(
        gathered.reshape(U, NG, GR, LANES), ufeatT, V, g2, bcol)
    return outT.transpose(2, 0, 1)


# speedup vs baseline: 1.3491x; 1.3491x over previous
import functools

import jax
import jax.numpy as jnp
from jax import lax
from jax.experimental import pallas as pl
from jax.experimental.pallas import tpu as pltpu
from jax.experimental.pallas import tpu_sc as plsc

B, U, NUM_UTYPE, NUM_UFEAT, EMB_DIM = 4096, 50, 1000000, 26, 32
BU = B * U

NC, NS = 2, 16
NW = NC * NS
LANES = 128
PACK = LANES // EMB_DIM


TBLK = 16384
QBLK = TBLK // PACK
NTBLK = pl.cdiv(NUM_UTYPE, TBLK)
RM_PAD_ROWS = NTBLK * QBLK


def _eye(n):
    a = lax.broadcasted_iota(jnp.int32, (n, n), 0)
    b = lax.broadcasted_iota(jnp.int32, (n, n), 1)
    return (a == b).astype(jnp.float32)


def _tr_body(in_ref, out_ref):
    x = in_ref[...]
    xs = jnp.concatenate(
        [x[:, q * QBLK:(q + 1) * QBLK] for q in range(PACK)], axis=0)
    out_ref[...] = jax.lax.dot_general(
        xs, _eye(LANES), dimension_numbers=(((0,), (0,)), ((), ())),
        preferred_element_type=jnp.float32,
    )


@jax.jit
def _tc_transpose(tableT):
    return pl.pallas_call(
        _tr_body,
        grid=(NTBLK,),
        in_specs=[pl.BlockSpec((EMB_DIM, TBLK), lambda k: (0, k))],
        out_specs=pl.BlockSpec((QBLK, LANES), lambda k: (k, 0)),
        out_shape=jax.ShapeDtypeStruct((RM_PAD_ROWS, LANES), jnp.float32),
    )(tableT)


ROWS_PER_W = BU // NW
CHUNK = 128
NCHUNK = ROWS_PER_W // CHUNK


def _sc_gather_body(table_hbm, idx_hbm, out_hbm, idx_v,
                    ridx0, ridx1, g0, g1, sem0, sem1):
    wid = lax.axis_index("s") * NC + lax.axis_index("c")
    base = wid * ROWS_PER_W
    pltpu.sync_copy(idx_hbm.at[pl.ds(wid * NCHUNK, NCHUNK)], idx_v)

    def start(j, ridx, buf, sem):
        for m in range(CHUNK // 16):
            iv = idx_v[j, pl.ds(m * 16, 16)]
            r32 = (lax.shift_left(lax.shift_right_logical(iv, 14), 14)
                   | lax.shift_left(iv & 4095, 2)
                   | (lax.shift_right_logical(iv, 12) & 3))
            ridx[pl.ds(m * 16, 16)] = r32
        pltpu.make_async_copy(table_hbm.at[ridx], buf, sem).start()

    def drain(j, ridx, buf, sem):
        pltpu.make_async_copy(table_hbm.at[ridx], buf, sem).wait()
        pltpu.sync_copy(buf, out_hbm.at[pl.ds(base + j * CHUNK, CHUNK)])

    start(0, ridx0, g0, sem0)
    start(1, ridx1, g1, sem1)

    def body(h, carry):
        j0 = 2 * h
        drain(j0, ridx0, g0, sem0)

        @pl.when(j0 + 2 < NCHUNK)
        def _():
            start(j0 + 2, ridx0, g0, sem0)

        drain(j0 + 1, ridx1, g1, sem1)

        @pl.when(j0 + 3 < NCHUNK)
        def _():
            start(j0 + 3, ridx1, g1, sem1)

        return carry

    lax.fori_loop(0, NCHUNK // 2, body, 0)


@jax.jit
def _sc_gather(tableRM, idx2):
    mesh = plsc.VectorSubcoreMesh(core_axis_name="c", subcore_axis_name="s")
    fn = pl.kernel(
        _sc_gather_body,
        out_type=jax.ShapeDtypeStruct((BU, EMB_DIM), jnp.float32),
        mesh=mesh,
        scratch_types=[
            pltpu.VMEM((NCHUNK, CHUNK), jnp.int32),
            pltpu.VMEM((CHUNK,), jnp.int32),
            pltpu.VMEM((CHUNK,), jnp.int32),
            pltpu.VMEM((CHUNK, EMB_DIM), jnp.float32),
            pltpu.VMEM((CHUNK, EMB_DIM), jnp.float32),
            pltpu.SemaphoreType.DMA,
            pltpu.SemaphoreType.DMA,
        ],
        compiler_params=pltpu.CompilerParams(use_tc_tiling_on_sc=False),
    )
    return fn(tableRM.reshape(RM_PAD_ROWS * PACK, EMB_DIM), idx2)


CB = 256
GR = CB // PACK
NG = B // CB


def _cmb_body(g_ref, u_ref, v_ref, gn_ref, b_ref, out_ref):
    v = v_ref[...]
    scale = gn_ref[0, 0] * lax.rsqrt(jnp.sum(v * v))
    w = v * scale
    bias = b_ref[...]
    eye = _eye(GR)
    ai = lax.broadcasted_iota(jnp.int32, (CB, CB), 0)
    bi = lax.broadcasted_iota(jnp.int32, (CB, CB), 1)
    perm = (bi == ((ai & (GR - 1)) * PACK
                   + lax.shift_right_logical(ai, 6))).astype(jnp.float32)
    xg_all = jnp.concatenate([g_ref[u, 0] for u in range(U)], axis=1)
    xgT_all = jax.lax.dot_general(
        xg_all, eye, dimension_numbers=(((0,), (0,)), ((), ())),
        preferred_element_type=jnp.float32,
    )
    gtps = []
    mms = []
    for u in range(U):
        x = u_ref[:, u, :]
        mms.append(jax.lax.dot_general(
            w, x, dimension_numbers=(((1,), (0,)), ((), ())),
            preferred_element_type=jnp.float32,
        ) + bias)
        xgT = xgT_all[u * LANES:(u + 1) * LANES, :]
        gtps.append(jnp.concatenate(
            [xgT[q * EMB_DIM:(q + 1) * EMB_DIM, :] for q in range(PACK)],
            axis=1))
    gt_all = jax.lax.dot_general(
        jnp.concatenate(gtps, axis=0), perm,
        dimension_numbers=(((1,), (0,)), ((), ())),
        preferred_element_type=jnp.float32,
    )
    for u in range(U):
        out_ref[u] = jnp.concatenate(
            [gt_all[u * EMB_DIM:(u + 1) * EMB_DIM, :], mms[u]], axis=0)


@jax.jit
def _tc_combine(gathered4, ufeatT, V, g2, bcol):
    return pl.pallas_call(
        _cmb_body,
        grid=(B // CB,),
        in_specs=[
            pl.BlockSpec((U, 1, GR, LANES), lambda c: (0, c, 0, 0)),
            pl.BlockSpec((NUM_UFEAT, U, CB), lambda c: (0, 0, c)),
            pl.BlockSpec((EMB_DIM, NUM_UFEAT), lambda c: (0, 0)),
            pl.BlockSpec((1, 1), lambda c: (0, 0), memory_space=pltpu.SMEM),
            pl.BlockSpec((EMB_DIM, 1), lambda c: (0, 0)),
        ],
        out_specs=pl.BlockSpec((U, 2 * EMB_DIM, CB), lambda c: (0, 0, c)),
        out_shape=jax.ShapeDtypeStruct((U, 2 * EMB_DIM, B), jnp.float32),
    )(gathered4, ufeatT, V, g2, bcol)


def kernel(utype, ufeat, table, V, g, b):
    tableT = table.T
    ufeatT = jnp.transpose(ufeat, (2, 1, 0))
    idxT = utype.T.astype(jnp.int32)
    idxf = lax.bitcast_convert_type(idxT, jnp.float32)
    idx_re = lax.bitcast_convert_type(
        idxf.reshape(BU // CHUNK, CHUNK), jnp.int32)

    tableRM = _tc_transpose(tableT)
    gathered = _sc_gather(tableRM, idx_re)
    g2 = jnp.reshape(g, (1, 1)).astype(jnp.float32)
    bcol = jnp.reshape(b, (EMB_DIM, 1))
    outT = _tc_combine(
        gathered.reshape(U, NG, GR, LANES), ufeatT, V, g2, bcol)
    return outT.transpose(2, 0, 1)


# speedup vs baseline: 1.4598x; 1.0821x over previous
import functools

import jax
import jax.numpy as jnp
from jax import lax
from jax.experimental import pallas as pl
from jax.experimental.pallas import tpu as pltpu
from jax.experimental.pallas import tpu_sc as plsc

B, U, NUM_UTYPE, NUM_UFEAT, EMB_DIM = 4096, 50, 1000000, 26, 32
BU = B * U

NC, NS = 2, 16
NW = NC * NS
LANES = 128
PACK = LANES // EMB_DIM


TBLK = 32768
TSH = 15
QSH = TSH - 2
QBLK = TBLK // PACK
NTBLK = pl.cdiv(NUM_UTYPE, TBLK)
RM_PAD_ROWS = NTBLK * QBLK


def _eye(n):
    a = lax.broadcasted_iota(jnp.int32, (n, n), 0)
    b = lax.broadcasted_iota(jnp.int32, (n, n), 1)
    return (a == b).astype(jnp.float32)


def _tr_body(in_ref, out_ref):
    x = in_ref[...]
    xs = jnp.concatenate(
        [x[:, q * QBLK:(q + 1) * QBLK] for q in range(PACK)], axis=0)
    out_ref[...] = jax.lax.dot_general(
        xs, _eye(LANES), dimension_numbers=(((0,), (0,)), ((), ())),
        preferred_element_type=jnp.float32,
    )


@jax.jit
def _tc_transpose(tableT):
    return pl.pallas_call(
        _tr_body,
        grid=(NTBLK,),
        in_specs=[pl.BlockSpec((EMB_DIM, TBLK), lambda k: (0, k))],
        out_specs=pl.BlockSpec((QBLK, LANES), lambda k: (k, 0)),
        out_shape=jax.ShapeDtypeStruct((RM_PAD_ROWS, LANES), jnp.float32),
    )(tableT)


ROWS_PER_W = BU // NW
CHUNK = 128
NCHUNK = ROWS_PER_W // CHUNK


def _sc_gather_body(table_hbm, idx_hbm, out_hbm, idx_v,
                    ridx0, ridx1, g0, g1, sem0, sem1):
    wid = lax.axis_index("s") * NC + lax.axis_index("c")
    base = wid * ROWS_PER_W
    pltpu.sync_copy(idx_hbm.at[pl.ds(wid * NCHUNK, NCHUNK)], idx_v)

    def start(j, ridx, buf, sem):
        for m in range(CHUNK // 16):
            iv = idx_v[j, pl.ds(m * 16, 16)]
            r32 = (lax.shift_left(lax.shift_right_logical(iv, TSH), TSH)
                   | lax.shift_left(iv & (QBLK - 1), 2)
                   | (lax.shift_right_logical(iv, QSH) & 3))
            ridx[pl.ds(m * 16, 16)] = r32
        pltpu.make_async_copy(table_hbm.at[ridx], buf, sem).start()

    def drain(j, ridx, buf, sem):
        pltpu.make_async_copy(table_hbm.at[ridx], buf, sem).wait()
        pltpu.sync_copy(buf, out_hbm.at[pl.ds(base + j * CHUNK, CHUNK)])

    start(0, ridx0, g0, sem0)
    start(1, ridx1, g1, sem1)

    def body(h, carry):
        j0 = 2 * h
        drain(j0, ridx0, g0, sem0)

        @pl.when(j0 + 2 < NCHUNK)
        def _():
            start(j0 + 2, ridx0, g0, sem0)

        drain(j0 + 1, ridx1, g1, sem1)

        @pl.when(j0 + 3 < NCHUNK)
        def _():
            start(j0 + 3, ridx1, g1, sem1)

        return carry

    lax.fori_loop(0, NCHUNK // 2, body, 0)


@jax.jit
def _sc_gather(tableRM, idx2):
    mesh = plsc.VectorSubcoreMesh(core_axis_name="c", subcore_axis_name="s")
    fn = pl.kernel(
        _sc_gather_body,
        out_type=jax.ShapeDtypeStruct((BU, EMB_DIM), jnp.float32),
        mesh=mesh,
        scratch_types=[
            pltpu.VMEM((NCHUNK, CHUNK), jnp.int32),
            pltpu.VMEM((CHUNK,), jnp.int32),
            pltpu.VMEM((CHUNK,), jnp.int32),
            pltpu.VMEM((CHUNK, EMB_DIM), jnp.float32),
            pltpu.VMEM((CHUNK, EMB_DIM), jnp.float32),
            pltpu.SemaphoreType.DMA,
            pltpu.SemaphoreType.DMA,
        ],
        compiler_params=pltpu.CompilerParams(use_tc_tiling_on_sc=False),
    )
    return fn(tableRM.reshape(RM_PAD_ROWS * PACK, EMB_DIM), idx2)


CB = 256
GR = CB // PACK
NG = B // CB


def _cmb_body(g_ref, u_ref, v_ref, gn_ref, b_ref, out_ref):
    v = v_ref[...]
    scale = gn_ref[0, 0] * lax.rsqrt(jnp.sum(v * v))
    w = v * scale
    bias = b_ref[...]
    eye = _eye(GR)
    ai = lax.broadcasted_iota(jnp.int32, (CB, CB), 0)
    bi = lax.broadcasted_iota(jnp.int32, (CB, CB), 1)
    perm = (bi == ((ai & (GR - 1)) * PACK
                   + lax.shift_right_logical(ai, 6))).astype(jnp.float32)
    xg_all = jnp.concatenate([g_ref[u, 0] for u in range(U)], axis=1)
    xgT_all = jax.lax.dot_general(
        xg_all, eye, dimension_numbers=(((0,), (0,)), ((), ())),
        preferred_element_type=jnp.float32,
    )
    gtps = []
    mms = []
    for u in range(U):
        x = u_ref[:, u, :]
        mms.append(jax.lax.dot_general(
            w, x, dimension_numbers=(((1,), (0,)), ((), ())),
            preferred_element_type=jnp.float32,
        ) + bias)
        xgT = xgT_all[u * LANES:(u + 1) * LANES, :]
        gtps.append(jnp.concatenate(
            [xgT[q * EMB_DIM:(q + 1) * EMB_DIM, :] for q in range(PACK)],
            axis=1))
    gt_all = jax.lax.dot_general(
        jnp.concatenate(gtps, axis=0), perm,
        dimension_numbers=(((1,), (0,)), ((), ())),
        preferred_element_type=jnp.float32,
    )
    for u in range(U):
        out_ref[u] = jnp.concatenate(
            [gt_all[u * EMB_DIM:(u + 1) * EMB_DIM, :], mms[u]], axis=0)


@jax.jit
def _tc_combine(gathered4, ufeatT, V, g2, bcol):
    return pl.pallas_call(
        _cmb_body,
        grid=(B // CB,),
        in_specs=[
            pl.BlockSpec((U, 1, GR, LANES), lambda c: (0, c, 0, 0)),
            pl.BlockSpec((NUM_UFEAT, U, CB), lambda c: (0, 0, c)),
            pl.BlockSpec((EMB_DIM, NUM_UFEAT), lambda c: (0, 0)),
            pl.BlockSpec((1, 1), lambda c: (0, 0), memory_space=pltpu.SMEM),
            pl.BlockSpec((EMB_DIM, 1), lambda c: (0, 0)),
        ],
        out_specs=pl.BlockSpec((U, 2 * EMB_DIM, CB), lambda c: (0, 0, c)),
        out_shape=jax.ShapeDtypeStruct((U, 2 * EMB_DIM, B), jnp.float32),
    )(gathered4, ufeatT, V, g2, bcol)


def kernel(utype, ufeat, table, V, g, b):
    tableT = table.T
    ufeatT = jnp.transpose(ufeat, (2, 1, 0))
    idxT = utype.T.astype(jnp.int32)
    idxf = lax.bitcast_convert_type(idxT, jnp.float32)
    idx_re = lax.bitcast_convert_type(
        idxf.reshape(BU // CHUNK, CHUNK), jnp.int32)

    tableRM = _tc_transpose(tableT)
    gathered = _sc_gather(tableRM, idx_re)
    g2 = jnp.reshape(g, (1, 1)).astype(jnp.float32)
    bcol = jnp.reshape(b, (EMB_DIM, 1))
    outT = _tc_combine(
        gathered.reshape(U, NG, GR, LANES), ufeatT, V, g2, bcol)
    return outT.transpose(2, 0, 1)


# speedup vs baseline: 1.4607x; 1.0006x over previous
import functools

import jax
import jax.numpy as jnp
from jax import lax
from jax.experimental import pallas as pl
from jax.experimental.pallas import tpu as pltpu
from jax.experimental.pallas import tpu_sc as plsc

B, U, NUM_UTYPE, NUM_UFEAT, EMB_DIM = 4096, 50, 1000000, 26, 32
BU = B * U

NC, NS = 2, 16
NW = NC * NS
LANES = 128
PACK = LANES // EMB_DIM


TBLK = 32768
TSH = 15
QSH = TSH - 2
QBLK = TBLK // PACK
NTBLK = pl.cdiv(NUM_UTYPE, TBLK)
RM_PAD_ROWS = NTBLK * QBLK


def _eye(n):
    a = lax.broadcasted_iota(jnp.int32, (n, n), 0)
    b = lax.broadcasted_iota(jnp.int32, (n, n), 1)
    return (a == b).astype(jnp.float32)


def _tr_body(in_ref, out_ref):
    x = in_ref[...]
    col = lax.broadcasted_iota(jnp.int32, (EMB_DIM, TBLK), 1)
    lim = NUM_UTYPE - pl.program_id(0) * TBLK
    x = jnp.where(col < lim, x, 0.0)
    xs = jnp.concatenate(
        [x[:, q * QBLK:(q + 1) * QBLK] for q in range(PACK)], axis=0)
    out_ref[...] = jax.lax.dot_general(
        xs, _eye(LANES), dimension_numbers=(((0,), (0,)), ((), ())),
        preferred_element_type=jnp.float32,
    )


@jax.jit
def _tc_transpose(tableT):
    return pl.pallas_call(
        _tr_body,
        grid=(NTBLK,),
        in_specs=[pl.BlockSpec((EMB_DIM, TBLK), lambda k: (0, k))],
        out_specs=pl.BlockSpec((QBLK, LANES), lambda k: (k, 0)),
        out_shape=jax.ShapeDtypeStruct((RM_PAD_ROWS, LANES), jnp.float32),
    )(tableT)


ROWS_PER_W = BU // NW
CHUNK = 128
NCHUNK = ROWS_PER_W // CHUNK


def _sc_gather_body(table_hbm, idx_hbm, out_hbm, idx_v,
                    ridx0, ridx1, g0, g1, sem0, sem1):
    wid = lax.axis_index("s") * NC + lax.axis_index("c")
    base = wid * ROWS_PER_W
    pltpu.sync_copy(idx_hbm.at[pl.ds(wid * NCHUNK, NCHUNK)], idx_v)

    def start(j, ridx, buf, sem):
        for m in range(CHUNK // 16):
            iv = idx_v[j, pl.ds(m * 16, 16)]
            r32 = (lax.shift_left(lax.shift_right_logical(iv, TSH), TSH)
                   | lax.shift_left(iv & (QBLK - 1), 2)
                   | (lax.shift_right_logical(iv, QSH) & 3))
            ridx[pl.ds(m * 16, 16)] = r32
        pltpu.make_async_copy(table_hbm.at[ridx], buf, sem).start()

    def drain(j, ridx, buf, sem):
        pltpu.make_async_copy(table_hbm.at[ridx], buf, sem).wait()
        pltpu.sync_copy(buf, out_hbm.at[pl.ds(base + j * CHUNK, CHUNK)])

    start(0, ridx0, g0, sem0)
    start(1, ridx1, g1, sem1)

    def body(h, carry):
        j0 = 2 * h
        drain(j0, ridx0, g0, sem0)

        @pl.when(j0 + 2 < NCHUNK)
        def _():
            start(j0 + 2, ridx0, g0, sem0)

        drain(j0 + 1, ridx1, g1, sem1)

        @pl.when(j0 + 3 < NCHUNK)
        def _():
            start(j0 + 3, ridx1, g1, sem1)

        return carry

    lax.fori_loop(0, NCHUNK // 2, body, 0)


@jax.jit
def _sc_gather(tableRM, idx2):
    mesh = plsc.VectorSubcoreMesh(core_axis_name="c", subcore_axis_name="s")
    fn = pl.kernel(
        _sc_gather_body,
        out_type=jax.ShapeDtypeStruct((BU, EMB_DIM), jnp.float32),
        mesh=mesh,
        scratch_types=[
            pltpu.VMEM((NCHUNK, CHUNK), jnp.int32),
            pltpu.VMEM((CHUNK,), jnp.int32),
            pltpu.VMEM((CHUNK,), jnp.int32),
            pltpu.VMEM((CHUNK, EMB_DIM), jnp.float32),
            pltpu.VMEM((CHUNK, EMB_DIM), jnp.float32),
            pltpu.SemaphoreType.DMA,
            pltpu.SemaphoreType.DMA,
        ],
        compiler_params=pltpu.CompilerParams(use_tc_tiling_on_sc=False),
    )
    return fn(tableRM.reshape(RM_PAD_ROWS * PACK, EMB_DIM), idx2)


CB = 256
GR = CB // PACK
NG = B // CB


def _cmb_body(g_ref, u_ref, v_ref, gn_ref, b_ref, out_ref):
    v = v_ref[...]
    scale = gn_ref[0, 0] * lax.rsqrt(jnp.sum(v * v))
    w = v * scale
    bias = b_ref[...]
    eye = _eye(GR)
    ai = lax.broadcasted_iota(jnp.int32, (CB, CB), 0)
    bi = lax.broadcasted_iota(jnp.int32, (CB, CB), 1)
    perm = (bi == ((ai & (GR - 1)) * PACK
                   + lax.shift_right_logical(ai, 6))).astype(jnp.float32)
    xg_all = jnp.concatenate([g_ref[u, 0] for u in range(U)], axis=1)
    xgT_all = jax.lax.dot_general(
        xg_all, eye, dimension_numbers=(((0,), (0,)), ((), ())),
        preferred_element_type=jnp.float32,
    )
    gtps = []
    mms = []
    for u in range(U):
        x = u_ref[:, u, :]
        mms.append(jax.lax.dot_general(
            w, x, dimension_numbers=(((1,), (0,)), ((), ())),
            preferred_element_type=jnp.float32,
        ) + bias)
        xgT = xgT_all[u * LANES:(u + 1) * LANES, :]
        gtps.append(jnp.concatenate(
            [xgT[q * EMB_DIM:(q + 1) * EMB_DIM, :] for q in range(PACK)],
            axis=1))
    gt_all = jax.lax.dot_general(
        jnp.concatenate(gtps, axis=0), perm,
        dimension_numbers=(((1,), (0,)), ((), ())),
        preferred_element_type=jnp.float32,
    )
    for u in range(U):
        out_ref[u] = jnp.concatenate(
            [gt_all[u * EMB_DIM:(u + 1) * EMB_DIM, :], mms[u]], axis=0)


@jax.jit
def _tc_combine(gathered4, ufeatT, V, g2, bcol):
    return pl.pallas_call(
        _cmb_body,
        grid=(B // CB,),
        in_specs=[
            pl.BlockSpec((U, 1, GR, LANES), lambda c: (0, c, 0, 0)),
            pl.BlockSpec((NUM_UFEAT, U, CB), lambda c: (0, 0, c)),
            pl.BlockSpec((EMB_DIM, NUM_UFEAT), lambda c: (0, 0)),
            pl.BlockSpec((1, 1), lambda c: (0, 0), memory_space=pltpu.SMEM),
            pl.BlockSpec((EMB_DIM, 1), lambda c: (0, 0)),
        ],
        out_specs=pl.BlockSpec((U, 2 * EMB_DIM, CB), lambda c: (0, 0, c)),
        out_shape=jax.ShapeDtypeStruct((U, 2 * EMB_DIM, B), jnp.float32),
    )(gathered4, ufeatT, V, g2, bcol)


def kernel(utype, ufeat, table, V, g, b):
    tableT = table.T
    ufeatT = jnp.transpose(ufeat, (2, 1, 0))
    idxT = utype.T.astype(jnp.int32)
    idxf = lax.bitcast_convert_type(idxT, jnp.float32)
    idx_re = lax.bitcast_convert_type(
        idxf.reshape(BU // CHUNK, CHUNK), jnp.int32)

    tableRM = _tc_transpose(tableT)
    gathered = _sc_gather(tableRM, idx_re)
    g2 = jnp.reshape(g, (1, 1)).astype(jnp.float32)
    bcol = jnp.reshape(b, (EMB_DIM, 1))
    outT = _tc_combine(
        gathered.reshape(U, NG, GR, LANES), ufeatT, V, g2, bcol)
    return outT.transpose(2, 0, 1)
